# baseline (speedup 1.0000x reference)
import functools

import jax
import jax.numpy as jnp
from jax import lax
from jax.experimental import pallas as pl
from jax.experimental.pallas import tpu as pltpu
from jax.experimental.pallas import tpu_sc as plsc

CHUNK = 128
SCH = 3
BUFR = SCH * CHUNK
NC = 2
NS = 16
NW = NC * NS


def _build(H, B, NSUP, first, last):
    HI = H // NS
    H2 = H // NC
    NCH = NSUP * SCH
    mesh = plsc.VectorSubcoreMesh(
        core_axis_name="c", subcore_axis_name="s", num_cores=NC)

    out_type = [jax.ShapeDtypeStruct((H, B), jnp.float32)]
    if not last:
        out_type += [
            jax.ShapeDtypeStruct((NC, H, B), jnp.float32),
            jax.ShapeDtypeStruct((NC, H, B), jnp.float32),
        ]

    scratch = [
        pltpu.VMEM((HI, B), jnp.float32),
        pltpu.VMEM((HI, B), jnp.float32),
    ]
    if not last:
        scratch += [
            pltpu.VMEM_SHARED((H, B), jnp.float32),
            pltpu.VMEM((NCH, CHUNK), jnp.int32),
            pltpu.VMEM((NCH, CHUNK), jnp.int32),
            pltpu.VMEM((NCH * CHUNK,), jnp.float32),
            pltpu.VMEM((2, BUFR, B), jnp.float32),
            pltpu.SemaphoreType.DMA,
            pltpu.SemaphoreType.DMA,
            pltpu.SemaphoreType.DMA,
            pltpu.SemaphoreType.DMA,
        ]

    def body(*refs):
        if first:
            (xb_hbm, cols_hbm, rows_hbm, vals_hbm,
             hs_hbm, hcop_hbm, part_hbm,
             abuf, bbuf, acc_sh, cols_v, rows_v, vals_v, gbuf,
             gsem0, gsem1, ssem0, ssem1) = refs
            pp_hbm = None
        elif last:
            (xb_hbm, pp_hbm, hs_hbm, abuf, bbuf) = refs
        else:
            (xb_hbm, cols_hbm, rows_hbm, vals_hbm, pp_hbm,
             hs_hbm, hcop_hbm, part_hbm,
             abuf, bbuf, acc_sh, cols_v, rows_v, vals_v, gbuf,
             gsem0, gsem1, ssem0, ssem1) = refs

        cid = lax.axis_index("c")
        sid = lax.axis_index("s")
        wid = cid * NS + sid
        ibase = sid * HI

        if not last:
            gsems = (gsem0, gsem1)
            ssems = (ssem0, ssem1)
            pltpu.sync_copy(cols_hbm.at[wid], cols_v)
            pltpu.sync_copy(rows_hbm.at[wid], rows_v)
            pltpu.sync_copy(vals_hbm.at[wid], vals_v)

        pltpu.sync_copy(xb_hbm.at[pl.ds(ibase, HI)], abuf)
        if pp_hbm is not None:
            pltpu.sync_copy(pp_hbm.at[0, pl.ds(ibase, HI)], bbuf)

            @plsc.parallel_loop(0, HI, unroll=4)
            def _add0(i):
                abuf[i, :] = abuf[i, :] + bbuf[i, :]

            pltpu.sync_copy(pp_hbm.at[1, pl.ds(ibase, HI)], bbuf)

            @plsc.parallel_loop(0, HI, unroll=4)
            def _add1(i):
                abuf[i, :] = abuf[i, :] + bbuf[i, :]

        @plsc.parallel_loop(0, HI, unroll=4)
        def _sig(i):
            abuf[i, :] = 1.0 / (1.0 + jnp.exp(-abuf[i, :]))

        own = jnp.logical_and(ibase >= cid * H2, ibase < (cid + 1) * H2)

        @pl.when(own)
        def _hs():
            pltpu.sync_copy(abuf, hs_hbm.at[pl.ds(ibase, HI)])

        if last:
            return

        pltpu.sync_copy(abuf, hcop_hbm.at[cid, pl.ds(ibase, HI)])

        @plsc.parallel_loop(0, HI, unroll=8)
        def _zero(i):
            bbuf[i, :] = jnp.zeros((16,), jnp.float32)

        pltpu.sync_copy(bbuf, acc_sh.at[pl.ds(ibase, HI)])
        plsc.subcore_barrier()

        hsrc = hcop_hbm.at[cid]

        def gather_descs(s, b):
            for c in range(SCH):
                yield (hsrc.at[cols_v.at[s * SCH + c]],
                       gbuf.at[b, pl.ds(c * CHUNK, CHUNK)], gsems[b])

        def scatter_descs(s, b):
            for c in range(SCH):
                yield (gbuf.at[b, pl.ds(c * CHUNK, CHUNK)],
                       acc_sh.at[rows_v.at[s * SCH + c]], ssems[b])

        def scale(s, b):
            @plsc.parallel_loop(0, BUFR // 16, unroll=4)
            def _grp(g):
                vals_vec = vals_v[pl.ds(s * BUFR + g * 16, 16)]
                for lane in range(16):
                    vv = jnp.full((16,), vals_vec[lane], jnp.float32)
                    r = g * 16 + lane
                    gbuf[b, r, :] = gbuf[b, r, :] * vv

        for sd in gather_descs(0, 0):
            pltpu.async_copy(*sd)

        @pl.loop(0, NSUP, step=2)
        def _sup(s0):
            for b in range(2):
                s = s0 + b
                nb = 1 - b

                @pl.when(s >= 1)
                def _drain_scatter():
                    for sd in scatter_descs(s, nb):
                        pltpu.make_async_copy(*sd).wait()

                @pl.when(s + 1 < NSUP)
                def _next_gather():
                    for sd in gather_descs(s + 1, nb):
                        pltpu.async_copy(*sd)

                for sd in gather_descs(s, b):
                    pltpu.make_async_copy(*sd).wait()
                scale(s, b)
                for sd in scatter_descs(s, b):
                    pltpu.async_copy(*sd, add=True)

        for sd in scatter_descs(NSUP - 1, 1):
            pltpu.make_async_copy(*sd).wait()
        plsc.subcore_barrier()

        pltpu.sync_copy(acc_sh.at[pl.ds(ibase, HI)],
                        part_hbm.at[cid, pl.ds(ibase, HI)])

    return functools.partial(
        pl.kernel, out_type=out_type, mesh=mesh,
        compiler_params=pltpu.CompilerParams(use_tc_tiling_on_sc=False),
        scratch_types=scratch)(body)


def kernel(x, hh_indices, hh_values, bias_hh):
    B, T, H = x.shape
    NNZ = hh_values.shape[0]
    per = -(-NNZ // NW)
    NSUP = -(-per // BUFR)
    NSUP += NSUP % 2
    cap = NW * NSUP * BUFR
    pad = cap - NNZ
    NCH = NSUP * SCH

    rows = jnp.concatenate([hh_indices[0], jnp.zeros((pad,), jnp.int32)])
    cols = jnp.concatenate([hh_indices[1], jnp.zeros((pad,), jnp.int32)])
    vals = jnp.concatenate([hh_values, jnp.zeros((pad,), jnp.float32)])
    rows = rows.reshape(NW, NCH, CHUNK)
    cols = cols.reshape(NW, NCH, CHUNK)
    vals = vals.reshape(NW, NCH * CHUNK)

    xb = jnp.transpose(x, (1, 2, 0)) + bias_hh[None]

    k_first = _build(H, B, NSUP, first=True, last=False)
    k_mid = _build(H, B, NSUP, first=False, last=False)
    k_last = _build(H, B, NSUP, first=False, last=True)

    hs_list = []
    hs_t, _, part = k_first(xb[0], cols, rows, vals)
    hs_list.append(hs_t)
    for t in range(1, T - 1):
        hs_t, _, part = k_mid(xb[t], cols, rows, vals, part)
        hs_list.append(hs_t)
    hs_list.append(k_last(xb[T - 1], part)[0])

    hs = jnp.stack(hs_list)
    return jnp.transpose(hs, (2, 0, 1))


# speedup vs baseline: 1.1305x; 1.1305x over previous
import functools

import jax
import jax.numpy as jnp
from jax import lax
from jax.experimental import pallas as pl
from jax.experimental.pallas import tpu as pltpu
from jax.experimental.pallas import tpu_sc as plsc

CHUNK = 128
SCH = 6
BUFR = SCH * CHUNK
NW = 16


def _build(T, H, B, NSUP):
    HP = H // NW
    NCH = NSUP * SCH
    mesh = plsc.VectorSubcoreMesh(
        core_axis_name="c", subcore_axis_name="s", num_cores=1)

    @functools.partial(
        pl.kernel,
        out_type=[
            jax.ShapeDtypeStruct((T, H, B), jnp.float32),
            jax.ShapeDtypeStruct((H, B), jnp.float32),
        ],
        mesh=mesh,
        compiler_params=pltpu.CompilerParams(use_tc_tiling_on_sc=False),
        scratch_types=[
            pltpu.VMEM_SHARED((H, B), jnp.float32),
            pltpu.VMEM((NCH, CHUNK), jnp.int32),
            pltpu.VMEM((NCH, CHUNK), jnp.int32),
            pltpu.VMEM((NCH * CHUNK,), jnp.float32),
            pltpu.VMEM((2, BUFR, B), jnp.float32),
            pltpu.VMEM((HP, B), jnp.float32),
            pltpu.SemaphoreType.DMA,
            pltpu.SemaphoreType.DMA,
            pltpu.SemaphoreType.DMA,
            pltpu.SemaphoreType.DMA,
        ],
    )
    def rnn(xb_hbm, cols_hbm, rows_hbm, vals_hbm, hs_hbm, h_hbm,
            acc_sh, cols_v, rows_v, vals_v, gbuf, pbuf,
            gsem0, gsem1, ssem0, ssem1):
        wid = lax.axis_index("s")
        rbase = wid * HP
        gsems = (gsem0, gsem1)
        ssems = (ssem0, ssem1)
        pltpu.sync_copy(cols_hbm.at[wid], cols_v)
        pltpu.sync_copy(rows_hbm.at[wid], rows_v)
        pltpu.sync_copy(vals_hbm.at[wid], vals_v)

        def gather_descs(s, b):
            for c in range(SCH):
                yield (h_hbm.at[cols_v.at[s * SCH + c]],
                       gbuf.at[b, pl.ds(c * CHUNK, CHUNK)], gsems[b])

        def scatter_descs(s, b):
            for c in range(SCH):
                yield (gbuf.at[b, pl.ds(c * CHUNK, CHUNK)],
                       acc_sh.at[rows_v.at[s * SCH + c]], ssems[b])

        def scale(s, b):
            @plsc.parallel_loop(0, BUFR // 16, unroll=2)
            def _grp(g):
                vals_vec = vals_v[pl.ds(s * BUFR + g * 16, 16)]
                for lane in range(16):
                    vv = jnp.full((16,), vals_vec[lane], jnp.float32)
                    r = g * 16 + lane
                    gbuf[b, r, :] = gbuf[b, r, :] * vv

        @pl.loop(0, T)
        def _step(t):
            @pl.when(t > 0)
            def _prefetch():
                for sd in gather_descs(0, 0):
                    pltpu.async_copy(*sd)

            pltpu.sync_copy(xb_hbm.at[t, pl.ds(rbase, HP)],
                            acc_sh.at[pl.ds(rbase, HP)])
            plsc.subcore_barrier()

            @pl.when(t > 0)
            def _spmm():
                @pl.loop(0, NSUP, step=2)
                def _sup(s0):
                    for b in range(2):
                        s = s0 + b
                        nb = 1 - b

                        @pl.when(s >= 1)
                        def _drain_scatter():
                            for sd in scatter_descs(s, nb):
                                pltpu.make_async_copy(*sd).wait()

                        @pl.when(s + 1 < NSUP)
                        def _next_gather():
                            for sd in gather_descs(s + 1, nb):
                                pltpu.async_copy(*sd)

                        for sd in gather_descs(s, b):
                            pltpu.make_async_copy(*sd).wait()
                        scale(s, b)
                        for sd in scatter_descs(s, b):
                            pltpu.async_copy(*sd, add=True)

                for sd in scatter_descs(NSUP - 1, 1):
                    pltpu.make_async_copy(*sd).wait()

            plsc.subcore_barrier()

            pltpu.sync_copy(acc_sh.at[pl.ds(rbase, HP)], pbuf)

            @plsc.parallel_loop(0, HP, unroll=4)
            def _pw(i):
                v = pbuf[i, :]
                pbuf[i, :] = 1.0 / (1.0 + jnp.exp(-v))

            pltpu.sync_copy(pbuf, h_hbm.at[pl.ds(rbase, HP)])
            pltpu.sync_copy(pbuf, hs_hbm.at[t, pl.ds(rbase, HP)])
            plsc.subcore_barrier()

    return rnn


def kernel(x, hh_indices, hh_values, bias_hh):
    B, T, H = x.shape
    NNZ = hh_values.shape[0]
    per = -(-NNZ // NW)
    NSUP = -(-per // BUFR)
    NSUP += NSUP % 2
    cap = NW * NSUP * BUFR
    pad = cap - NNZ
    NCH = NSUP * SCH

    rows = jnp.concatenate([hh_indices[0], jnp.zeros((pad,), jnp.int32)])
    cols = jnp.concatenate([hh_indices[1], jnp.zeros((pad,), jnp.int32)])
    vals = jnp.concatenate([hh_values, jnp.zeros((pad,), jnp.float32)])
    rows = rows.reshape(NW, NCH, CHUNK)
    cols = cols.reshape(NW, NCH, CHUNK)
    vals = vals.reshape(NW, NCH * CHUNK)

    xb = jnp.transpose(x, (1, 2, 0)) + bias_hh[None]

    hs, _ = _build(T, H, B, NSUP)(xb, cols, rows, vals)
    return jnp.transpose(hs, (2, 0, 1))


# speedup vs baseline: 1.1380x; 1.0066x over previous
import functools

import jax
import jax.numpy as jnp
from jax import lax
from jax.experimental import pallas as pl
from jax.experimental.pallas import tpu as pltpu
from jax.experimental.pallas import tpu_sc as plsc

CHUNK = 128
SCH = 6
BUFR = SCH * CHUNK
NW = 16


def _build(T, H, B, NSUP):
    HP = H // NW
    NCH = NSUP * SCH
    mesh = plsc.VectorSubcoreMesh(
        core_axis_name="c", subcore_axis_name="s", num_cores=1)

    @functools.partial(
        pl.kernel,
        out_type=[
            jax.ShapeDtypeStruct((T, H, B), jnp.float32),
        ],
        mesh=mesh,
        compiler_params=pltpu.CompilerParams(use_tc_tiling_on_sc=False),
        scratch_types=[
            pltpu.VMEM_SHARED((H, B), jnp.float32),
            pltpu.VMEM((NCH, CHUNK), jnp.int32),
            pltpu.VMEM((NCH, CHUNK), jnp.int32),
            pltpu.VMEM((NCH * CHUNK,), jnp.float32),
            pltpu.VMEM((2, BUFR, B), jnp.float32),
            pltpu.VMEM((HP, B), jnp.float32),
            pltpu.SemaphoreType.DMA,
            pltpu.SemaphoreType.DMA,
            pltpu.SemaphoreType.DMA,
            pltpu.SemaphoreType.DMA,
        ],
    )
    def rnn(xb_hbm, cols_hbm, rows_hbm, vals_hbm, hs_hbm,
            acc_sh, cols_v, rows_v, vals_v, gbuf, pbuf,
            gsem0, gsem1, ssem0, ssem1):
        wid = lax.axis_index("s")
        rbase = wid * HP
        gsems = (gsem0, gsem1)
        ssems = (ssem0, ssem1)
        pltpu.sync_copy(cols_hbm.at[wid], cols_v)
        pltpu.sync_copy(rows_hbm.at[wid], rows_v)
        pltpu.sync_copy(vals_hbm.at[wid], vals_v)

        def gather_descs(t, s, b):
            hprev = hs_hbm.at[t - 1]
            for c in range(SCH):
                yield (hprev.at[cols_v.at[s * SCH + c]],
                       gbuf.at[b, pl.ds(c * CHUNK, CHUNK)], gsems[b])

        def scatter_descs(s, b):
            for c in range(SCH):
                yield (gbuf.at[b, pl.ds(c * CHUNK, CHUNK)],
                       acc_sh.at[rows_v.at[s * SCH + c]], ssems[b])

        def scale(s, b):
            @plsc.parallel_loop(0, BUFR // 16, unroll=2)
            def _grp(g):
                vals_vec = vals_v[pl.ds(s * BUFR + g * 16, 16)]
                for lane in range(16):
                    vv = jnp.full((16,), vals_vec[lane], jnp.float32)
                    r = g * 16 + lane
                    gbuf[b, r, :] = gbuf[b, r, :] * vv

        @pl.loop(0, T)
        def _step(t):
            @pl.when(t > 0)
            def _prefetch():
                for sd in gather_descs(t, 0, 0):
                    pltpu.async_copy(*sd)

            pltpu.sync_copy(xb_hbm.at[t, pl.ds(rbase, HP)],
                            acc_sh.at[pl.ds(rbase, HP)])
            plsc.subcore_barrier()

            @pl.when(t > 0)
            def _spmm():
                @pl.loop(0, NSUP, step=2)
                def _sup(s0):
                    for b in range(2):
                        s = s0 + b
                        nb = 1 - b

                        @pl.when(s >= 1)
                        def _drain_scatter():
                            for sd in scatter_descs(s, nb):
                                pltpu.make_async_copy(*sd).wait()

                        @pl.when(s + 1 < NSUP)
                        def _next_gather():
                            for sd in gather_descs(t, s + 1, nb):
                                pltpu.async_copy(*sd)

                        for sd in gather_descs(t, s, b):
                            pltpu.make_async_copy(*sd).wait()
                        scale(s, b)
                        for sd in scatter_descs(s, b):
                            pltpu.async_copy(*sd, add=True)

                for sd in scatter_descs(NSUP - 1, 1):
                    pltpu.make_async_copy(*sd).wait()

            plsc.subcore_barrier()

            pltpu.sync_copy(acc_sh.at[pl.ds(rbase, HP)], pbuf)

            @plsc.parallel_loop(0, HP, unroll=4)
            def _pw(i):
                v = pbuf[i, :]
                pbuf[i, :] = 1.0 / (1.0 + jnp.exp(-v))

            pltpu.sync_copy(pbuf, hs_hbm.at[t, pl.ds(rbase, HP)])
            plsc.subcore_barrier()

    return rnn


def kernel(x, hh_indices, hh_values, bias_hh):
    B, T, H = x.shape
    NNZ = hh_values.shape[0]
    per = -(-NNZ // NW)
    NSUP = -(-per // BUFR)
    NSUP += NSUP % 2
    cap = NW * NSUP * BUFR
    pad = cap - NNZ
    NCH = NSUP * SCH

    rows = jnp.concatenate([hh_indices[0], jnp.zeros((pad,), jnp.int32)])
    cols = jnp.concatenate([hh_indices[1], jnp.zeros((pad,), jnp.int32)])
    vals = jnp.concatenate([hh_values, jnp.zeros((pad,), jnp.float32)])
    rows = rows.reshape(NW, NCH, CHUNK)
    cols = cols.reshape(NW, NCH, CHUNK)
    vals = vals.reshape(NW, NCH * CHUNK)

    xb = jnp.transpose(x, (1, 2, 0)) + bias_hh[None]

    (hs,) = _build(T, H, B, NSUP)(xb, cols, rows, vals)
    return jnp.transpose(hs, (2, 0, 1))
